# Initial kernel scaffold; baseline (speedup 1.0000x reference)
import functools

import jax
import jax.numpy as jnp
from jax import lax
from jax.experimental import pallas as pl
from jax.experimental.pallas import tpu as pltpu
from jax.experimental.pallas import tpu_sc as plsc

_B, _J, _H, _W = 128, 17, 96, 96
_HW = _H * _W
_ROWS = _B * _J
_N = _ROWS * _HW
_K1 = _N // 2
_K2 = _K1 + 1

_NC, _NS, _L = 2, 16, 16
_NW = _NC * _NS
_PER_W = _N // _NW
_CHUNK = 4096
_NCHUNKS = _PER_W // _CHUNK

_BINS12 = 2048
_BINS3 = 1024

_mesh = plsc.VectorSubcoreMesh(
    core_axis_name="c", subcore_axis_name="s", num_cores=_NC, num_subcores=_NS
)


_R = 64


def _loss_body(o_ref, t_ref, w_ref, out_ref):
    w = w_ref[...]
    d = o_ref[...] * w - t_ref[...] * w
    out_ref[...] = 0.5 * (d * d)


def _loss_tc(o2, t2, w2):
    return pl.pallas_call(
        _loss_body,
        grid=(_ROWS // _R,),
        in_specs=[
            pl.BlockSpec((_R, _HW), lambda i: (i, 0)),
            pl.BlockSpec((_R, _HW), lambda i: (i, 0)),
            pl.BlockSpec((_R, 1), lambda i: (i, 0)),
        ],
        out_specs=pl.BlockSpec((_R, _HW), lambda i: (i, 0)),
        out_shape=jax.ShapeDtypeStruct((_ROWS, _HW), jnp.float32),
    )(o2, t2, w2)


def _make_sc_pass(npass):
    nbins = _BINS12 if npass < 3 else _BINS3
    nsec = 1 if npass == 1 else 2
    hw = nsec * _L * nbins
    bin_shift = {1: 21, 2: 10, 3: 0}[npass]
    pref_shift = {2: 21, 3: 10}.get(npass)

    scratch = [
        pltpu.VMEM((_CHUNK,), jnp.int32),
        pltpu.VMEM((hw,), jnp.int32),
    ]
    if npass > 1:
        scratch.append(pltpu.VMEM((1024,), jnp.int32))

    @functools.partial(
        pl.kernel,
        out_type=jax.ShapeDtypeStruct((_NW, hw), jnp.int32),
        mesh=_mesh,
        scratch_types=scratch,
    )
    def sc_hist(bits_hbm, *args):
        if npass > 1:
            sel_hbm, hist_hbm, buf, hist_v, sel_v = args
        else:
            hist_hbm, buf, hist_v = args
        cid = lax.axis_index("c")
        sid = lax.axis_index("s")
        wid = sid * _NC + cid
        lane = lax.iota(jnp.int32, _L)
        ones = jnp.ones((_L,), jnp.int32)
        zeros = jnp.zeros((_L,), jnp.int32)

        def zero_body(i, carry):
            hist_v[pl.ds(i * _L, _L)] = zeros
            return carry

        lax.fori_loop(0, hw // _L, zero_body, 0)

        if npass > 1:
            pltpu.sync_copy(sel_hbm, sel_v)
            sel_a = sel_v[pl.ds(0, _L)]
            sel_b = sel_v[pl.ds(128, _L)]

        lane_base = lane * nbins

        def chunk_body(ci, carry):
            base = wid * _PER_W + ci * _CHUNK
            pltpu.sync_copy(bits_hbm.at[pl.ds(base, _CHUNK)], buf)

            def vec_body(j, c2):
                v = buf[pl.ds(j * _L, _L)]
                bins = jnp.right_shift(v, bin_shift) & (nbins - 1)
                idx = lane_base + bins
                if npass == 1:
                    plsc.addupdate_scatter(hist_v, [idx], ones)
                else:
                    pref = jnp.right_shift(v, pref_shift)
                    plsc.addupdate_scatter(hist_v, [idx], ones, mask=pref == sel_a)
                    plsc.addupdate_scatter(
                        hist_v, [idx + _L * nbins], ones, mask=pref == sel_b
                    )
                return c2

            lax.fori_loop(0, _CHUNK // _L, vec_body, 0)
            return carry

        lax.fori_loop(0, _NCHUNKS, chunk_body, 0)
        pltpu.sync_copy(hist_v, hist_hbm.at[wid])

    return sc_hist


_sc_pass1 = _make_sc_pass(1)
_sc_pass2 = _make_sc_pass(2)
_sc_pass3 = _make_sc_pass(3)


def _cumsum_lanes(x):
    m = x.shape[1]
    s = 1
    while s < m:
        x = x + jnp.concatenate(
            [jnp.zeros((1, s), x.dtype), x[:, : m - s]], axis=1
        )
        s *= 2
    return x


def _row_val(arr, row):
    r = lax.broadcasted_iota(jnp.int32, arr.shape, 0)
    return jnp.max(jnp.where(r == row, arr, 0))


def _pack_rows(vals, dtype):
    r = lax.broadcasted_iota(jnp.int32, (8, 128), 0)
    out = jnp.zeros((8, 128), dtype)
    for i, v in enumerate(vals):
        out = jnp.where(r == i, v, out)
    return out


def _find(cum, rank):
    below = cum < rank
    b = jnp.sum(below.astype(jnp.int32))
    cnt = jnp.max(jnp.where(below, cum, 0))
    return b, cnt


def _sel1_body(h_ref, out_ref):
    h = h_ref[...]
    tot = jnp.sum(h, axis=0, keepdims=True)
    cum = _cumsum_lanes(tot)
    bin_a, cnt_a = _find(cum, _K1)
    bin_b, cnt_b = _find(cum, _K2)
    out_ref[...] = _pack_rows([bin_a, bin_b, cnt_a, cnt_b], jnp.int32)


def _sel23_body(h_ref, sel_ref, out_ref, *, nbins, final):
    h = h_ref[...]
    rows = lax.broadcasted_iota(jnp.int32, h.shape, 0)
    is_a = ((rows // _L) % 2) == 0
    tot_a = jnp.sum(jnp.where(is_a, h, 0), axis=0, keepdims=True)
    tot_b = jnp.sum(jnp.where(is_a, 0, h), axis=0, keepdims=True)
    cum_a = _cumsum_lanes(tot_a)
    cum_b = _cumsum_lanes(tot_b)
    sel = sel_ref[...]
    pref_a = _row_val(sel, 0)
    pref_b = _row_val(sel, 1)
    cnt_a = _row_val(sel, 2)
    cnt_b = _row_val(sel, 3)
    bin2_a, cnt2_a = _find(cum_a, _K1 - cnt_a)
    bin2_b, cnt2_b = _find(cum_b, _K2 - cnt_b)
    npref_a = pref_a * nbins + bin2_a
    npref_b = pref_b * nbins + bin2_b
    if not final:
        out_ref[...] = _pack_rows(
            [npref_a, npref_b, cnt_a + cnt2_a, cnt_b + cnt2_b], jnp.int32
        )
    else:
        a = lax.bitcast_convert_type(
            jnp.full((1, 1), npref_a, jnp.int32), jnp.float32
        )
        b = lax.bitcast_convert_type(
            jnp.full((1, 1), npref_b, jnp.int32), jnp.float32
        )
        med = jnp.broadcast_to((a + b) * 0.5, (8, 128))
        r = lax.broadcasted_iota(jnp.int32, (8, 128), 0)
        out_ref[...] = jnp.where(r == 0, med, 0.0)


def _select1(h):
    return pl.pallas_call(
        _sel1_body,
        out_shape=jax.ShapeDtypeStruct((8, 128), jnp.int32),
    )(h)


def _select23(h, sel, nbins, final):
    body = functools.partial(_sel23_body, nbins=nbins, final=final)
    return pl.pallas_call(
        body,
        out_shape=jax.ShapeDtypeStruct(
            (8, 128), jnp.float32 if final else jnp.int32
        ),
    )(h, sel)


def _final_body(loss_ref, med_ref, out_ref, acc_s, acc_c):
    i = pl.program_id(0)
    med = jnp.max(med_ref[...])
    x = loss_ref[...]
    m = x > med
    s = jnp.sum(jnp.where(m, x, 0.0))
    c = jnp.sum(m.astype(jnp.int32))

    @pl.when(i == 0)
    def _():
        acc_s[0] = s
        acc_c[0] = c

    @pl.when(i > 0)
    def _():
        acc_s[0] += s
        acc_c[0] += c

    @pl.when(i == pl.num_programs(0) - 1)
    def _():
        out_ref[...] = jnp.full(
            (8, 128), acc_s[0] / acc_c[0].astype(jnp.float32), jnp.float32
        )


def _final_tc(loss, med):
    return pl.pallas_call(
        _final_body,
        grid=(_ROWS // _R,),
        in_specs=[
            pl.BlockSpec((_R, _HW), lambda i: (i, 0)),
            pl.BlockSpec((8, 128), lambda i: (0, 0)),
        ],
        out_specs=pl.BlockSpec((8, 128), lambda i: (0, 0)),
        out_shape=jax.ShapeDtypeStruct((8, 128), jnp.float32),
        scratch_shapes=[
            pltpu.SMEM((1,), jnp.float32),
            pltpu.SMEM((1,), jnp.int32),
        ],
    )(loss, med)


def kernel(output, target, target_weight):
    o2 = output.reshape(_ROWS, _HW)
    t2 = target.reshape(_ROWS, _HW)
    w2 = target_weight.reshape(_ROWS, 1)

    loss = _loss_tc(o2, t2, w2)
    bits = lax.bitcast_convert_type(loss, jnp.int32).reshape(_N)

    h1 = _sc_pass1(bits)
    sel1 = _select1(h1.reshape(_NW * _L, _BINS12))
    h2 = _sc_pass2(bits, sel1.reshape(-1))
    sel2 = _select23(h2.reshape(_NW * 2 * _L, _BINS12), sel1, _BINS12, False)
    h3 = _sc_pass3(bits, sel2.reshape(-1))
    med = _select23(h3.reshape(_NW * 2 * _L, _BINS3), sel2, _BINS3, True)

    res = _final_tc(loss, med)
    return res[0, 0]


# speedup vs baseline: 19.6065x; 19.6065x over previous
import functools

import jax
import jax.numpy as jnp
from jax import lax
from jax.experimental import pallas as pl
from jax.experimental.pallas import tpu as pltpu
from jax.experimental.pallas import tpu_sc as plsc

_B, _J, _H, _W = 128, 17, 96, 96
_HW = _H * _W
_ROWS = _B * _J
_N = _ROWS * _HW
_K1 = _N // 2
_K2 = _K1 + 1

_NC, _NS, _L = 2, 16, 16
_NW = _NC * _NS
_PER_W = _N // _NW
_CHUNK = 4096
_NCHUNKS = _PER_W // _CHUNK

_BINS12 = 2048
_BINS3 = 1024

_mesh = plsc.VectorSubcoreMesh(
    core_axis_name="c", subcore_axis_name="s", num_cores=_NC, num_subcores=_NS
)


_R = 64


def _loss_body(o_ref, t_ref, w_ref, out_ref):
    w = w_ref[...]
    d = o_ref[...] * w - t_ref[...] * w
    out_ref[...] = 0.5 * (d * d)


def _loss_tc(o2, t2, w2):
    return pl.pallas_call(
        _loss_body,
        grid=(_ROWS // _R,),
        in_specs=[
            pl.BlockSpec((_R, _HW), lambda i: (i, 0)),
            pl.BlockSpec((_R, _HW), lambda i: (i, 0)),
            pl.BlockSpec((_R, 1), lambda i: (i, 0)),
        ],
        out_specs=pl.BlockSpec((_R, _HW), lambda i: (i, 0)),
        out_shape=jax.ShapeDtypeStruct((_ROWS, _HW), jnp.float32),
    )(o2, t2, w2)


def _make_sc_pass(npass):
    nbins = _BINS12 if npass < 3 else _BINS3
    nsec = 1 if npass == 1 else 2
    hw = nsec * _L * nbins
    bin_shift = {1: 21, 2: 10, 3: 0}[npass]
    pref_shift = {2: 21, 3: 10}.get(npass)

    scratch = [
        pltpu.VMEM((_CHUNK,), jnp.int32),
        pltpu.VMEM((hw,), jnp.int32),
    ]
    if npass > 1:
        scratch.append(pltpu.VMEM((1024,), jnp.int32))

    @functools.partial(
        pl.kernel,
        out_type=jax.ShapeDtypeStruct((_NW, hw), jnp.int32),
        mesh=_mesh,
        scratch_types=scratch,
        compiler_params=pltpu.CompilerParams(needs_layout_passes=False),
    )
    def sc_hist(bits_hbm, *args):
        if npass > 1:
            sel_hbm, hist_hbm, buf, hist_v, sel_v = args
        else:
            hist_hbm, buf, hist_v = args
        cid = lax.axis_index("c")
        sid = lax.axis_index("s")
        wid = sid * _NC + cid
        lane = lax.iota(jnp.int32, _L)
        ones = jnp.ones((_L,), jnp.int32)
        zeros = jnp.zeros((_L,), jnp.int32)

        def zero_body(i, carry):
            hist_v[pl.ds(i * _L, _L)] = zeros
            return carry

        lax.fori_loop(0, hw // _L, zero_body, 0)

        if npass > 1:
            pltpu.sync_copy(sel_hbm, sel_v)
            sel_a = sel_v[pl.ds(0, _L)]
            sel_b = sel_v[pl.ds(128, _L)]

        lane_base = lane * nbins

        def chunk_body(ci, carry):
            base = wid * _PER_W + ci * _CHUNK
            pltpu.sync_copy(bits_hbm.at[pl.ds(base, _CHUNK)], buf)

            def vec_body(j, c2):
                v = buf[pl.ds(j * _L, _L)]
                bins = jnp.right_shift(v, bin_shift) & (nbins - 1)
                idx = lane_base + bins
                if npass == 1:
                    plsc.addupdate_scatter(hist_v, [idx], ones)
                else:
                    pref = jnp.right_shift(v, pref_shift)
                    plsc.addupdate_scatter(hist_v, [idx], ones, mask=pref == sel_a)
                    plsc.addupdate_scatter(
                        hist_v, [idx + _L * nbins], ones, mask=pref == sel_b
                    )
                return c2

            lax.fori_loop(0, _CHUNK // _L, vec_body, 0)
            return carry

        lax.fori_loop(0, _NCHUNKS, chunk_body, 0)
        pltpu.sync_copy(hist_v, hist_hbm.at[wid])

    return sc_hist


_sc_pass1 = _make_sc_pass(1)
_sc_pass2 = _make_sc_pass(2)
_sc_pass3 = _make_sc_pass(3)


def _cumsum_lanes(x):
    m = x.shape[1]
    s = 1
    while s < m:
        x = x + jnp.concatenate(
            [jnp.zeros((1, s), x.dtype), x[:, : m - s]], axis=1
        )
        s *= 2
    return x


def _row_val(arr, row):
    r = lax.broadcasted_iota(jnp.int32, arr.shape, 0)
    return jnp.max(jnp.where(r == row, arr, 0))


def _pack_rows(vals, dtype):
    r = lax.broadcasted_iota(jnp.int32, (8, 128), 0)
    out = jnp.zeros((8, 128), dtype)
    for i, v in enumerate(vals):
        out = jnp.where(r == i, v, out)
    return out


def _find(cum, rank):
    below = cum < rank
    b = jnp.sum(below.astype(jnp.int32))
    cnt = jnp.max(jnp.where(below, cum, 0))
    return b, cnt


def _sel1_body(h_ref, out_ref):
    h = h_ref[...]
    tot = jnp.sum(h, axis=0, keepdims=True)
    cum = _cumsum_lanes(tot)
    bin_a, cnt_a = _find(cum, _K1)
    bin_b, cnt_b = _find(cum, _K2)
    out_ref[...] = _pack_rows([bin_a, bin_b, cnt_a, cnt_b], jnp.int32)


def _sel23_body(h_ref, sel_ref, out_ref, *, nbins, final):
    h = h_ref[...]
    rows = lax.broadcasted_iota(jnp.int32, h.shape, 0)
    is_a = ((rows // _L) % 2) == 0
    tot_a = jnp.sum(jnp.where(is_a, h, 0), axis=0, keepdims=True)
    tot_b = jnp.sum(jnp.where(is_a, 0, h), axis=0, keepdims=True)
    cum_a = _cumsum_lanes(tot_a)
    cum_b = _cumsum_lanes(tot_b)
    sel = sel_ref[...]
    pref_a = _row_val(sel, 0)
    pref_b = _row_val(sel, 1)
    cnt_a = _row_val(sel, 2)
    cnt_b = _row_val(sel, 3)
    bin2_a, cnt2_a = _find(cum_a, _K1 - cnt_a)
    bin2_b, cnt2_b = _find(cum_b, _K2 - cnt_b)
    npref_a = pref_a * nbins + bin2_a
    npref_b = pref_b * nbins + bin2_b
    if not final:
        out_ref[...] = _pack_rows(
            [npref_a, npref_b, cnt_a + cnt2_a, cnt_b + cnt2_b], jnp.int32
        )
    else:
        a = lax.bitcast_convert_type(
            jnp.full((1, 1), npref_a, jnp.int32), jnp.float32
        )
        b = lax.bitcast_convert_type(
            jnp.full((1, 1), npref_b, jnp.int32), jnp.float32
        )
        med = jnp.broadcast_to((a + b) * 0.5, (8, 128))
        r = lax.broadcasted_iota(jnp.int32, (8, 128), 0)
        out_ref[...] = jnp.where(r == 0, med, 0.0)


def _select1(h):
    return pl.pallas_call(
        _sel1_body,
        out_shape=jax.ShapeDtypeStruct((8, 128), jnp.int32),
    )(h)


def _select23(h, sel, nbins, final):
    body = functools.partial(_sel23_body, nbins=nbins, final=final)
    return pl.pallas_call(
        body,
        out_shape=jax.ShapeDtypeStruct(
            (8, 128), jnp.float32 if final else jnp.int32
        ),
    )(h, sel)


def _final_body(loss_ref, med_ref, out_ref, acc_s, acc_c):
    i = pl.program_id(0)
    med = jnp.max(med_ref[...])
    x = loss_ref[...]
    m = x > med
    s = jnp.sum(jnp.where(m, x, 0.0))
    c = jnp.sum(m.astype(jnp.int32))

    @pl.when(i == 0)
    def _():
        acc_s[0] = s
        acc_c[0] = c

    @pl.when(i > 0)
    def _():
        acc_s[0] += s
        acc_c[0] += c

    @pl.when(i == pl.num_programs(0) - 1)
    def _():
        out_ref[...] = jnp.full(
            (8, 128), acc_s[0] / acc_c[0].astype(jnp.float32), jnp.float32
        )


def _final_tc(loss, med):
    return pl.pallas_call(
        _final_body,
        grid=(_ROWS // _R,),
        in_specs=[
            pl.BlockSpec((_R, _HW), lambda i: (i, 0)),
            pl.BlockSpec((8, 128), lambda i: (0, 0)),
        ],
        out_specs=pl.BlockSpec((8, 128), lambda i: (0, 0)),
        out_shape=jax.ShapeDtypeStruct((8, 128), jnp.float32),
        scratch_shapes=[
            pltpu.SMEM((1,), jnp.float32),
            pltpu.SMEM((1,), jnp.int32),
        ],
    )(loss, med)


def kernel(output, target, target_weight):
    o2 = output.reshape(_ROWS, _HW)
    t2 = target.reshape(_ROWS, _HW)
    w2 = target_weight.reshape(_ROWS, 1)

    loss = _loss_tc(o2, t2, w2)
    bits = lax.bitcast_convert_type(loss, jnp.int32).reshape(_N)

    h1 = _sc_pass1(bits)
    sel1 = _select1(h1.reshape(_NW * _L, _BINS12))
    h2 = _sc_pass2(bits, sel1.reshape(-1))
    sel2 = _select23(h2.reshape(_NW * 2 * _L, _BINS12), sel1, _BINS12, False)
    h3 = _sc_pass3(bits, sel2.reshape(-1))
    med = _select23(h3.reshape(_NW * 2 * _L, _BINS3), sel2, _BINS3, True)

    res = _final_tc(loss, med)
    return res[0, 0]


# speedup vs baseline: 24.9143x; 1.2707x over previous
import functools

import jax
import jax.numpy as jnp
from jax import lax
from jax.experimental import pallas as pl
from jax.experimental.pallas import tpu as pltpu
from jax.experimental.pallas import tpu_sc as plsc

_B, _J, _H, _W = 128, 17, 96, 96
_HW = _H * _W
_ROWS = _B * _J
_N = _ROWS * _HW
_K1 = _N // 2
_K2 = _K1 + 1

_NC, _NS, _L = 2, 16, 16
_NW = _NC * _NS
_PER_W = _N // _NW
_CHUNK = 6144
_NCHUNKS = _PER_W // _CHUNK
_UNROLL = 4

_BINS12 = 2048
_BINS3 = 1024

_mesh = plsc.VectorSubcoreMesh(
    core_axis_name="c", subcore_axis_name="s", num_cores=_NC, num_subcores=_NS
)


_R = 64


def _loss_body(o_ref, t_ref, w_ref, out_ref):
    w = w_ref[...]
    d = o_ref[...] * w - t_ref[...] * w
    out_ref[...] = 0.5 * (d * d)


def _loss_tc(o2, t2, w2):
    return pl.pallas_call(
        _loss_body,
        grid=(_ROWS // _R,),
        in_specs=[
            pl.BlockSpec((_R, _HW), lambda i: (i, 0)),
            pl.BlockSpec((_R, _HW), lambda i: (i, 0)),
            pl.BlockSpec((_R, 1), lambda i: (i, 0)),
        ],
        out_specs=pl.BlockSpec((_R, _HW), lambda i: (i, 0)),
        out_shape=jax.ShapeDtypeStruct((_ROWS, _HW), jnp.float32),
    )(o2, t2, w2)


def _make_sc_pass(npass):
    nbins = _BINS12 if npass < 3 else _BINS3
    nsec = 1 if npass == 1 else 2
    hw = nsec * _L * nbins
    bin_shift = {1: 21, 2: 10, 3: 0}[npass]
    pref_shift = {2: 21, 3: 10}.get(npass)

    scratch = [
        pltpu.VMEM((_CHUNK,), jnp.int32),
        pltpu.VMEM((_CHUNK,), jnp.int32),
        pltpu.VMEM((hw,), jnp.int32),
        pltpu.SemaphoreType.DMA,
        pltpu.SemaphoreType.DMA,
    ]
    if npass > 1:
        scratch.append(pltpu.VMEM((1024,), jnp.int32))

    @functools.partial(
        pl.kernel,
        out_type=jax.ShapeDtypeStruct((_NW, hw), jnp.int32),
        mesh=_mesh,
        scratch_types=scratch,
        compiler_params=pltpu.CompilerParams(needs_layout_passes=False),
    )
    def sc_hist(bits_hbm, *args):
        if npass > 1:
            sel_hbm, hist_hbm, buf0, buf1, hist_v, sem0, sem1, sel_v = args
        else:
            hist_hbm, buf0, buf1, hist_v, sem0, sem1 = args
        bufs = (buf0, buf1)
        sems = (sem0, sem1)
        cid = lax.axis_index("c")
        sid = lax.axis_index("s")
        wid = sid * _NC + cid
        lane = lax.iota(jnp.int32, _L)
        ones = jnp.ones((_L,), jnp.int32)
        zeros = jnp.zeros((_L,), jnp.int32)

        def zero_body(i, carry):
            hist_v[pl.ds(i * _L, _L)] = zeros
            return carry

        lax.fori_loop(0, hw // _L, zero_body, 0)

        if npass > 1:
            pltpu.sync_copy(sel_hbm, sel_v)
            sel_a = sel_v[pl.ds(0, _L)]
            sel_b = sel_v[pl.ds(128, _L)]

        lane_base = lane * nbins
        w_base = wid * _PER_W

        def copy_desc(ci, b):
            return pltpu.make_async_copy(
                bits_hbm.at[pl.ds(w_base + ci * _CHUNK, _CHUNK)], bufs[b], sems[b]
            )

        def process(buf):
            def vec_body(j, c2):
                for u in range(_UNROLL):
                    v = buf[pl.ds((j * _UNROLL + u) * _L, _L)]
                    bins = jnp.right_shift(v, bin_shift) & (nbins - 1)
                    idx = lane_base + bins
                    if npass == 1:
                        plsc.addupdate_scatter(hist_v, [idx], ones)
                    else:
                        pref = jnp.right_shift(v, pref_shift)
                        plsc.addupdate_scatter(
                            hist_v, [idx], ones, mask=pref == sel_a
                        )
                        plsc.addupdate_scatter(
                            hist_v, [idx + _L * nbins], ones, mask=pref == sel_b
                        )
                return c2

            lax.fori_loop(0, _CHUNK // (_L * _UNROLL), vec_body, 0)

        copy_desc(0, 0).start()
        copy_desc(1, 1).start()

        def chunk_body(g, carry):
            for b in range(2):
                ci = g * 2 + b
                copy_desc(ci, b).wait()
                process(bufs[b])
                copy_desc(ci + 2, b).start()
            return carry

        lax.fori_loop(0, _NCHUNKS // 2 - 1, chunk_body, 0)
        for b in range(2):
            ci = _NCHUNKS - 2 + b
            copy_desc(ci, b).wait()
            process(bufs[b])
        pltpu.sync_copy(hist_v, hist_hbm.at[wid])

    return sc_hist


_sc_pass1 = _make_sc_pass(1)
_sc_pass2 = _make_sc_pass(2)
_sc_pass3 = _make_sc_pass(3)


def _cumsum_lanes(x):
    m = x.shape[1]
    s = 1
    while s < m:
        x = x + jnp.concatenate(
            [jnp.zeros((1, s), x.dtype), x[:, : m - s]], axis=1
        )
        s *= 2
    return x


def _row_val(arr, row):
    r = lax.broadcasted_iota(jnp.int32, arr.shape, 0)
    return jnp.max(jnp.where(r == row, arr, 0))


def _pack_rows(vals, dtype):
    r = lax.broadcasted_iota(jnp.int32, (8, 128), 0)
    out = jnp.zeros((8, 128), dtype)
    for i, v in enumerate(vals):
        out = jnp.where(r == i, v, out)
    return out


def _find(cum, rank):
    below = cum < rank
    b = jnp.sum(below.astype(jnp.int32))
    cnt = jnp.max(jnp.where(below, cum, 0))
    return b, cnt


def _sel1_body(h_ref, out_ref):
    h = h_ref[...]
    tot = jnp.sum(h, axis=0, keepdims=True)
    cum = _cumsum_lanes(tot)
    bin_a, cnt_a = _find(cum, _K1)
    bin_b, cnt_b = _find(cum, _K2)
    out_ref[...] = _pack_rows([bin_a, bin_b, cnt_a, cnt_b], jnp.int32)


def _sel23_body(h_ref, sel_ref, out_ref, *, nbins, final):
    h = h_ref[...]
    rows = lax.broadcasted_iota(jnp.int32, h.shape, 0)
    is_a = ((rows // _L) % 2) == 0
    tot_a = jnp.sum(jnp.where(is_a, h, 0), axis=0, keepdims=True)
    tot_b = jnp.sum(jnp.where(is_a, 0, h), axis=0, keepdims=True)
    cum_a = _cumsum_lanes(tot_a)
    cum_b = _cumsum_lanes(tot_b)
    sel = sel_ref[...]
    pref_a = _row_val(sel, 0)
    pref_b = _row_val(sel, 1)
    cnt_a = _row_val(sel, 2)
    cnt_b = _row_val(sel, 3)
    bin2_a, cnt2_a = _find(cum_a, _K1 - cnt_a)
    bin2_b, cnt2_b = _find(cum_b, _K2 - cnt_b)
    npref_a = pref_a * nbins + bin2_a
    npref_b = pref_b * nbins + bin2_b
    if not final:
        out_ref[...] = _pack_rows(
            [npref_a, npref_b, cnt_a + cnt2_a, cnt_b + cnt2_b], jnp.int32
        )
    else:
        a = lax.bitcast_convert_type(
            jnp.full((1, 1), npref_a, jnp.int32), jnp.float32
        )
        b = lax.bitcast_convert_type(
            jnp.full((1, 1), npref_b, jnp.int32), jnp.float32
        )
        med = jnp.broadcast_to((a + b) * 0.5, (8, 128))
        r = lax.broadcasted_iota(jnp.int32, (8, 128), 0)
        out_ref[...] = jnp.where(r == 0, med, 0.0)


def _select1(h):
    return pl.pallas_call(
        _sel1_body,
        out_shape=jax.ShapeDtypeStruct((8, 128), jnp.int32),
    )(h)


def _select23(h, sel, nbins, final):
    body = functools.partial(_sel23_body, nbins=nbins, final=final)
    return pl.pallas_call(
        body,
        out_shape=jax.ShapeDtypeStruct(
            (8, 128), jnp.float32 if final else jnp.int32
        ),
    )(h, sel)


def _final_body(loss_ref, med_ref, out_ref, acc_s, acc_c):
    i = pl.program_id(0)
    med = jnp.max(med_ref[...])
    x = loss_ref[...]
    m = x > med
    s = jnp.sum(jnp.where(m, x, 0.0))
    c = jnp.sum(m.astype(jnp.int32))

    @pl.when(i == 0)
    def _():
        acc_s[0] = s
        acc_c[0] = c

    @pl.when(i > 0)
    def _():
        acc_s[0] += s
        acc_c[0] += c

    @pl.when(i == pl.num_programs(0) - 1)
    def _():
        out_ref[...] = jnp.full(
            (8, 128), acc_s[0] / acc_c[0].astype(jnp.float32), jnp.float32
        )


def _final_tc(loss, med):
    return pl.pallas_call(
        _final_body,
        grid=(_ROWS // _R,),
        in_specs=[
            pl.BlockSpec((_R, _HW), lambda i: (i, 0)),
            pl.BlockSpec((8, 128), lambda i: (0, 0)),
        ],
        out_specs=pl.BlockSpec((8, 128), lambda i: (0, 0)),
        out_shape=jax.ShapeDtypeStruct((8, 128), jnp.float32),
        scratch_shapes=[
            pltpu.SMEM((1,), jnp.float32),
            pltpu.SMEM((1,), jnp.int32),
        ],
    )(loss, med)


def kernel(output, target, target_weight):
    o2 = output.reshape(_ROWS, _HW)
    t2 = target.reshape(_ROWS, _HW)
    w2 = target_weight.reshape(_ROWS, 1)

    loss = _loss_tc(o2, t2, w2)
    bits = lax.bitcast_convert_type(loss, jnp.int32).reshape(_N)

    h1 = _sc_pass1(bits)
    sel1 = _select1(h1.reshape(_NW * _L, _BINS12))
    h2 = _sc_pass2(bits, sel1.reshape(-1))
    sel2 = _select23(h2.reshape(_NW * 2 * _L, _BINS12), sel1, _BINS12, False)
    h3 = _sc_pass3(bits, sel2.reshape(-1))
    med = _select23(h3.reshape(_NW * 2 * _L, _BINS3), sel2, _BINS3, True)

    res = _final_tc(loss, med)
    return res[0, 0]


# speedup vs baseline: 25.6524x; 1.0296x over previous
import functools

import jax
import jax.numpy as jnp
from jax import lax
from jax.experimental import pallas as pl
from jax.experimental.pallas import tpu as pltpu
from jax.experimental.pallas import tpu_sc as plsc

_B, _J, _H, _W = 128, 17, 96, 96
_HW = _H * _W
_ROWS = _B * _J
_N = _ROWS * _HW
_K1 = _N // 2
_K2 = _K1 + 1

_NC, _NS, _L = 2, 16, 16
_NW = _NC * _NS
_PER_W = _N // _NW
_CHUNK = 6144
_NCHUNKS = _PER_W // _CHUNK
_UNROLL = 4

_BINS12 = 2048
_BINS3 = 1024

_mesh = plsc.VectorSubcoreMesh(
    core_axis_name="c", subcore_axis_name="s", num_cores=_NC, num_subcores=_NS
)


_R = 64


def _loss_body(o_ref, t_ref, w_ref, out_ref):
    w = w_ref[...]
    d = o_ref[...] * w - t_ref[...] * w
    out_ref[...] = lax.bitcast_convert_type(0.5 * (d * d), jnp.int32)


def _loss_tc(o2, t2, w2):
    return pl.pallas_call(
        _loss_body,
        grid=(_ROWS // _R,),
        in_specs=[
            pl.BlockSpec((_R, _HW), lambda i: (i, 0)),
            pl.BlockSpec((_R, _HW), lambda i: (i, 0)),
            pl.BlockSpec((_R, 1), lambda i: (i, 0)),
        ],
        out_specs=pl.BlockSpec((_R, _HW), lambda i: (i, 0)),
        out_shape=jax.ShapeDtypeStruct((_ROWS, _HW), jnp.int32),
    )(o2, t2, w2)


def _make_sc_pass(npass):
    nbins = _BINS12 if npass < 3 else _BINS3
    nsec = 1 if npass == 1 else 2
    hw = nsec * _L * nbins
    bin_shift = {1: 21, 2: 10, 3: 0}[npass]
    pref_shift = {2: 21, 3: 10}.get(npass)

    scratch = [
        pltpu.VMEM((_CHUNK,), jnp.int32),
        pltpu.VMEM((_CHUNK,), jnp.int32),
        pltpu.VMEM((hw,), jnp.int32),
        pltpu.SemaphoreType.DMA,
        pltpu.SemaphoreType.DMA,
    ]
    if npass > 1:
        scratch.append(pltpu.VMEM((1024,), jnp.int32))

    @functools.partial(
        pl.kernel,
        out_type=jax.ShapeDtypeStruct((_NW, hw), jnp.int32),
        mesh=_mesh,
        scratch_types=scratch,
        compiler_params=pltpu.CompilerParams(needs_layout_passes=False),
    )
    def sc_hist(bits_hbm, *args):
        if npass > 1:
            sel_hbm, hist_hbm, buf0, buf1, hist_v, sem0, sem1, sel_v = args
        else:
            hist_hbm, buf0, buf1, hist_v, sem0, sem1 = args
        bufs = (buf0, buf1)
        sems = (sem0, sem1)
        cid = lax.axis_index("c")
        sid = lax.axis_index("s")
        wid = sid * _NC + cid
        lane = lax.iota(jnp.int32, _L)
        ones = jnp.ones((_L,), jnp.int32)
        zeros = jnp.zeros((_L,), jnp.int32)

        def zero_body(i, carry):
            hist_v[pl.ds(i * _L, _L)] = zeros
            return carry

        lax.fori_loop(0, hw // _L, zero_body, 0)

        if npass > 1:
            pltpu.sync_copy(sel_hbm, sel_v)
            sel_a = sel_v[pl.ds(0, _L)]
            sel_b = sel_v[pl.ds(128, _L)]

        lane_base = lane * nbins
        w_base = wid * _PER_W

        def copy_desc(ci, b):
            return pltpu.make_async_copy(
                bits_hbm.at[pl.ds(w_base + ci * _CHUNK, _CHUNK)], bufs[b], sems[b]
            )

        def process(buf):
            def vec_body(j, c2):
                for u in range(_UNROLL):
                    v = buf[pl.ds((j * _UNROLL + u) * _L, _L)]
                    bins = jnp.right_shift(v, bin_shift) & (nbins - 1)
                    idx = lane_base + bins
                    if npass == 1:
                        plsc.addupdate_scatter(hist_v, [idx], ones)
                    else:
                        pref = jnp.right_shift(v, pref_shift)
                        plsc.addupdate_scatter(
                            hist_v, [idx], ones, mask=pref == sel_a
                        )
                        plsc.addupdate_scatter(
                            hist_v, [idx + _L * nbins], ones, mask=pref == sel_b
                        )
                return c2

            lax.fori_loop(0, _CHUNK // (_L * _UNROLL), vec_body, 0)

        copy_desc(0, 0).start()
        copy_desc(1, 1).start()

        def chunk_body(g, carry):
            for b in range(2):
                ci = g * 2 + b
                copy_desc(ci, b).wait()
                process(bufs[b])
                copy_desc(ci + 2, b).start()
            return carry

        lax.fori_loop(0, _NCHUNKS // 2 - 1, chunk_body, 0)
        for b in range(2):
            ci = _NCHUNKS - 2 + b
            copy_desc(ci, b).wait()
            process(bufs[b])
        pltpu.sync_copy(hist_v, hist_hbm.at[wid])

    return sc_hist


_sc_pass1 = _make_sc_pass(1)
_sc_pass2 = _make_sc_pass(2)
_sc_pass3 = _make_sc_pass(3)


def _cumsum_lanes(x):
    m = x.shape[1]
    s = 1
    while s < m:
        x = x + jnp.concatenate(
            [jnp.zeros((1, s), x.dtype), x[:, : m - s]], axis=1
        )
        s *= 2
    return x


def _row_val(arr, row):
    r = lax.broadcasted_iota(jnp.int32, arr.shape, 0)
    return jnp.max(jnp.where(r == row, arr, 0))


def _pack_rows(vals, dtype):
    r = lax.broadcasted_iota(jnp.int32, (8, 128), 0)
    out = jnp.zeros((8, 128), dtype)
    for i, v in enumerate(vals):
        out = jnp.where(r == i, v, out)
    return out


def _find(cum, rank):
    below = cum < rank
    b = jnp.sum(below.astype(jnp.int32))
    cnt = jnp.max(jnp.where(below, cum, 0))
    return b, cnt


def _sel1_body(h_ref, out_ref):
    h = h_ref[...]
    tot = jnp.sum(h, axis=0, keepdims=True)
    cum = _cumsum_lanes(tot)
    bin_a, cnt_a = _find(cum, _K1)
    bin_b, cnt_b = _find(cum, _K2)
    out_ref[...] = _pack_rows([bin_a, bin_b, cnt_a, cnt_b], jnp.int32)


def _sel23_body(h_ref, sel_ref, out_ref, *, nbins, final):
    h = h_ref[...]
    rows = lax.broadcasted_iota(jnp.int32, h.shape, 0)
    is_a = ((rows // _L) % 2) == 0
    tot_a = jnp.sum(jnp.where(is_a, h, 0), axis=0, keepdims=True)
    tot_b = jnp.sum(jnp.where(is_a, 0, h), axis=0, keepdims=True)
    cum_a = _cumsum_lanes(tot_a)
    cum_b = _cumsum_lanes(tot_b)
    sel = sel_ref[...]
    pref_a = _row_val(sel, 0)
    pref_b = _row_val(sel, 1)
    cnt_a = _row_val(sel, 2)
    cnt_b = _row_val(sel, 3)
    bin2_a, cnt2_a = _find(cum_a, _K1 - cnt_a)
    bin2_b, cnt2_b = _find(cum_b, _K2 - cnt_b)
    npref_a = pref_a * nbins + bin2_a
    npref_b = pref_b * nbins + bin2_b
    if not final:
        out_ref[...] = _pack_rows(
            [npref_a, npref_b, cnt_a + cnt2_a, cnt_b + cnt2_b], jnp.int32
        )
    else:
        a = lax.bitcast_convert_type(
            jnp.full((1, 1), npref_a, jnp.int32), jnp.float32
        )
        b = lax.bitcast_convert_type(
            jnp.full((1, 1), npref_b, jnp.int32), jnp.float32
        )
        med = jnp.broadcast_to((a + b) * 0.5, (8, 128))
        r = lax.broadcasted_iota(jnp.int32, (8, 128), 0)
        out_ref[...] = jnp.where(r == 0, med, 0.0)


def _select1(h):
    return pl.pallas_call(
        _sel1_body,
        out_shape=jax.ShapeDtypeStruct((8, 128), jnp.int32),
    )(h)


def _select23(h, sel, nbins, final):
    body = functools.partial(_sel23_body, nbins=nbins, final=final)
    return pl.pallas_call(
        body,
        out_shape=jax.ShapeDtypeStruct(
            (8, 128), jnp.float32 if final else jnp.int32
        ),
    )(h, sel)


def _final_body(loss_ref, med_ref, out_ref, acc_s, acc_c):
    i = pl.program_id(0)
    med = jnp.max(med_ref[...])
    x = lax.bitcast_convert_type(loss_ref[...], jnp.float32)
    m = x > med
    s = jnp.sum(jnp.where(m, x, 0.0))
    c = jnp.sum(m.astype(jnp.int32))

    @pl.when(i == 0)
    def _():
        acc_s[0] = s
        acc_c[0] = c

    @pl.when(i > 0)
    def _():
        acc_s[0] += s
        acc_c[0] += c

    @pl.when(i == pl.num_programs(0) - 1)
    def _():
        out_ref[...] = jnp.full(
            (8, 128), acc_s[0] / acc_c[0].astype(jnp.float32), jnp.float32
        )


def _final_tc(loss, med):
    return pl.pallas_call(
        _final_body,
        grid=(_ROWS // _R,),
        in_specs=[
            pl.BlockSpec((_R, _HW), lambda i: (i, 0)),
            pl.BlockSpec((8, 128), lambda i: (0, 0)),
        ],
        out_specs=pl.BlockSpec((8, 128), lambda i: (0, 0)),
        out_shape=jax.ShapeDtypeStruct((8, 128), jnp.float32),
        scratch_shapes=[
            pltpu.SMEM((1,), jnp.float32),
            pltpu.SMEM((1,), jnp.int32),
        ],
    )(loss, med)


def kernel(output, target, target_weight):
    o2 = output.reshape(_ROWS, _HW)
    t2 = target.reshape(_ROWS, _HW)
    w2 = target_weight.reshape(_ROWS, 1)

    bits2 = _loss_tc(o2, t2, w2)
    bits = bits2.reshape(_N)

    h1 = _sc_pass1(bits)
    sel1 = _select1(h1.reshape(_NW * _L, _BINS12))
    h2 = _sc_pass2(bits, sel1.reshape(-1))
    sel2 = _select23(h2.reshape(_NW * 2 * _L, _BINS12), sel1, _BINS12, False)
    h3 = _sc_pass3(bits, sel2.reshape(-1))
    med = _select23(h3.reshape(_NW * 2 * _L, _BINS3), sel2, _BINS3, True)

    res = _final_tc(bits2, med)
    return res[0, 0]


# speedup vs baseline: 28.5626x; 1.1134x over previous
import functools

import jax
import jax.numpy as jnp
from jax import lax
from jax.experimental import pallas as pl
from jax.experimental.pallas import tpu as pltpu
from jax.experimental.pallas import tpu_sc as plsc

_B, _J, _H, _W = 128, 17, 96, 96
_HW = _H * _W
_ROWS = _B * _J
_N = _ROWS * _HW
_K1 = _N // 2
_K2 = _K1 + 1

_NC, _NS, _L = 2, 16, 16
_NW = _NC * _NS
_RPW = (_B * _J * _H) // _NW
_CROWS = 64
_NCHUNKS = _RPW // _CROWS
_VPR = _W // _L

_BINS12 = 2048
_BINS3 = 1024

_mesh = plsc.VectorSubcoreMesh(
    core_axis_name="c", subcore_axis_name="s", num_cores=_NC, num_subcores=_NS
)


_MR = _B * _J * _H
_BB = 4
_BLKR = _BB * _J * _H


def _loss_body(o_ref, t_ref, w_ref, out_ref):
    w = w_ref[...][..., None]
    d = o_ref[...] * w - t_ref[...] * w
    bits = lax.bitcast_convert_type(0.5 * (d * d), jnp.int32)
    out_ref[...] = bits.reshape(_BLKR, _W)


def _loss_tc(o4, t4, w3):
    return pl.pallas_call(
        _loss_body,
        grid=(_B // _BB,),
        in_specs=[
            pl.BlockSpec((_BB, _J, _H, _W), lambda i: (i, 0, 0, 0)),
            pl.BlockSpec((_BB, _J, _H, _W), lambda i: (i, 0, 0, 0)),
            pl.BlockSpec((_BB, _J, 1), lambda i: (i, 0, 0)),
        ],
        out_specs=pl.BlockSpec((_BLKR, _W), lambda i: (i, 0)),
        out_shape=jax.ShapeDtypeStruct((_MR, _W), jnp.int32),
    )(o4, t4, w3)


def _make_sc_pass(npass):
    nbins = _BINS12 if npass < 3 else _BINS3
    nsec = 1 if npass == 1 else 2
    hw = nsec * _L * nbins
    bin_shift = {1: 21, 2: 10, 3: 0}[npass]
    pref_shift = {2: 21, 3: 10}.get(npass)

    scratch = [
        pltpu.VMEM((_CROWS, _W), jnp.int32),
        pltpu.VMEM((_CROWS, _W), jnp.int32),
        pltpu.VMEM((hw,), jnp.int32),
        pltpu.SemaphoreType.DMA,
        pltpu.SemaphoreType.DMA,
    ]
    if npass > 1:
        scratch.append(pltpu.VMEM((1024,), jnp.int32))

    @functools.partial(
        pl.kernel,
        out_type=jax.ShapeDtypeStruct((_NW, hw), jnp.int32),
        mesh=_mesh,
        scratch_types=scratch,
        compiler_params=pltpu.CompilerParams(needs_layout_passes=False),
    )
    def sc_hist(bits_hbm, *args):
        if npass > 1:
            sel_hbm, hist_hbm, buf0, buf1, hist_v, sem0, sem1, sel_v = args
        else:
            hist_hbm, buf0, buf1, hist_v, sem0, sem1 = args
        bufs = (buf0, buf1)
        sems = (sem0, sem1)
        cid = lax.axis_index("c")
        sid = lax.axis_index("s")
        wid = sid * _NC + cid
        lane = lax.iota(jnp.int32, _L)
        ones = jnp.ones((_L,), jnp.int32)
        zeros = jnp.zeros((_L,), jnp.int32)

        def zero_body(i, carry):
            hist_v[pl.ds(i * _L, _L)] = zeros
            return carry

        lax.fori_loop(0, hw // _L, zero_body, 0)

        if npass > 1:
            pltpu.sync_copy(sel_hbm, sel_v)
            sel_a = sel_v[pl.ds(0, _L)]
            sel_b = sel_v[pl.ds(128, _L)]

        lane_base = lane * nbins
        w_row0 = wid * _RPW

        def copy_desc(ci, b):
            return pltpu.make_async_copy(
                bits_hbm.at[pl.ds(w_row0 + ci * _CROWS, _CROWS), :],
                bufs[b],
                sems[b],
            )

        def process(buf):
            def row_body(r, c2):
                for u in range(_VPR):
                    v = buf[r, pl.ds(u * _L, _L)]
                    bins = jnp.right_shift(v, bin_shift) & (nbins - 1)
                    idx = lane_base + bins
                    if npass == 1:
                        plsc.addupdate_scatter(hist_v, [idx], ones)
                    else:
                        pref = jnp.right_shift(v, pref_shift)
                        plsc.addupdate_scatter(
                            hist_v, [idx], ones, mask=pref == sel_a
                        )
                        plsc.addupdate_scatter(
                            hist_v, [idx + _L * nbins], ones, mask=pref == sel_b
                        )
                return c2

            lax.fori_loop(0, _CROWS, row_body, 0)

        copy_desc(0, 0).start()
        copy_desc(1, 1).start()

        def chunk_body(g, carry):
            for b in range(2):
                ci = g * 2 + b
                copy_desc(ci, b).wait()
                process(bufs[b])
                copy_desc(ci + 2, b).start()
            return carry

        lax.fori_loop(0, _NCHUNKS // 2 - 1, chunk_body, 0)
        for b in range(2):
            ci = _NCHUNKS - 2 + b
            copy_desc(ci, b).wait()
            process(bufs[b])
        pltpu.sync_copy(hist_v, hist_hbm.at[wid])

    return sc_hist


_sc_pass1 = _make_sc_pass(1)
_sc_pass2 = _make_sc_pass(2)
_sc_pass3 = _make_sc_pass(3)


def _cumsum_lanes(x):
    m = x.shape[1]
    s = 1
    while s < m:
        x = x + jnp.concatenate(
            [jnp.zeros((1, s), x.dtype), x[:, : m - s]], axis=1
        )
        s *= 2
    return x


def _row_val(arr, row):
    r = lax.broadcasted_iota(jnp.int32, arr.shape, 0)
    return jnp.max(jnp.where(r == row, arr, 0))


def _pack_rows(vals, dtype):
    r = lax.broadcasted_iota(jnp.int32, (8, 128), 0)
    out = jnp.zeros((8, 128), dtype)
    for i, v in enumerate(vals):
        out = jnp.where(r == i, v, out)
    return out


def _find(cum, rank):
    below = cum < rank
    b = jnp.sum(below.astype(jnp.int32))
    cnt = jnp.max(jnp.where(below, cum, 0))
    return b, cnt


def _sel1_body(h_ref, out_ref):
    h = h_ref[...]
    tot = jnp.sum(h, axis=0, keepdims=True)
    cum = _cumsum_lanes(tot)
    bin_a, cnt_a = _find(cum, _K1)
    bin_b, cnt_b = _find(cum, _K2)
    out_ref[...] = _pack_rows([bin_a, bin_b, cnt_a, cnt_b], jnp.int32)


def _sel23_body(h_ref, sel_ref, out_ref, *, nbins, final):
    h = h_ref[...]
    rows = lax.broadcasted_iota(jnp.int32, h.shape, 0)
    is_a = ((rows // _L) % 2) == 0
    tot_a = jnp.sum(jnp.where(is_a, h, 0), axis=0, keepdims=True)
    tot_b = jnp.sum(jnp.where(is_a, 0, h), axis=0, keepdims=True)
    cum_a = _cumsum_lanes(tot_a)
    cum_b = _cumsum_lanes(tot_b)
    sel = sel_ref[...]
    pref_a = _row_val(sel, 0)
    pref_b = _row_val(sel, 1)
    cnt_a = _row_val(sel, 2)
    cnt_b = _row_val(sel, 3)
    bin2_a, cnt2_a = _find(cum_a, _K1 - cnt_a)
    bin2_b, cnt2_b = _find(cum_b, _K2 - cnt_b)
    npref_a = pref_a * nbins + bin2_a
    npref_b = pref_b * nbins + bin2_b
    if not final:
        out_ref[...] = _pack_rows(
            [npref_a, npref_b, cnt_a + cnt2_a, cnt_b + cnt2_b], jnp.int32
        )
    else:
        a = lax.bitcast_convert_type(
            jnp.full((1, 1), npref_a, jnp.int32), jnp.float32
        )
        b = lax.bitcast_convert_type(
            jnp.full((1, 1), npref_b, jnp.int32), jnp.float32
        )
        med = jnp.broadcast_to((a + b) * 0.5, (8, 128))
        r = lax.broadcasted_iota(jnp.int32, (8, 128), 0)
        out_ref[...] = jnp.where(r == 0, med, 0.0)


def _select1(h):
    return pl.pallas_call(
        _sel1_body,
        out_shape=jax.ShapeDtypeStruct((8, 128), jnp.int32),
    )(h)


def _select23(h, sel, nbins, final):
    body = functools.partial(_sel23_body, nbins=nbins, final=final)
    return pl.pallas_call(
        body,
        out_shape=jax.ShapeDtypeStruct(
            (8, 128), jnp.float32 if final else jnp.int32
        ),
    )(h, sel)


def _final_body(loss_ref, med_ref, out_ref, acc_s, acc_c):
    i = pl.program_id(0)
    med = jnp.max(med_ref[...])
    x = lax.bitcast_convert_type(loss_ref[...], jnp.float32)
    m = x > med
    s = jnp.sum(jnp.where(m, x, 0.0))
    c = jnp.sum(m.astype(jnp.int32))

    @pl.when(i == 0)
    def _():
        acc_s[0] = s
        acc_c[0] = c

    @pl.when(i > 0)
    def _():
        acc_s[0] += s
        acc_c[0] += c

    @pl.when(i == pl.num_programs(0) - 1)
    def _():
        out_ref[...] = jnp.full(
            (8, 128), acc_s[0] / acc_c[0].astype(jnp.float32), jnp.float32
        )


def _final_tc(bits2, med):
    return pl.pallas_call(
        _final_body,
        grid=(_MR // _BLKR,),
        in_specs=[
            pl.BlockSpec((_BLKR, _W), lambda i: (i, 0)),
            pl.BlockSpec((8, 128), lambda i: (0, 0)),
        ],
        out_specs=pl.BlockSpec((8, 128), lambda i: (0, 0)),
        out_shape=jax.ShapeDtypeStruct((8, 128), jnp.float32),
        scratch_shapes=[
            pltpu.SMEM((1,), jnp.float32),
            pltpu.SMEM((1,), jnp.int32),
        ],
    )(bits2, med)


def kernel(output, target, target_weight):
    bits = _loss_tc(output, target, target_weight)

    h1 = _sc_pass1(bits)
    sel1 = _select1(h1.reshape(_NW * _L, _BINS12))
    h2 = _sc_pass2(bits, sel1.reshape(-1))
    sel2 = _select23(h2.reshape(_NW * 2 * _L, _BINS12), sel1, _BINS12, False)
    h3 = _sc_pass3(bits, sel2.reshape(-1))
    med = _select23(h3.reshape(_NW * 2 * _L, _BINS3), sel2, _BINS3, True)

    res = _final_tc(bits, med)
    return res[0, 0]


# speedup vs baseline: 31.3874x; 1.0989x over previous
import functools

import jax
import jax.numpy as jnp
from jax import lax
from jax.experimental import pallas as pl
from jax.experimental.pallas import tpu as pltpu
from jax.experimental.pallas import tpu_sc as plsc

_B, _J, _H, _W = 128, 17, 96, 96
_HW = _H * _W
_ROWS = _B * _J
_N = _ROWS * _HW
_K1 = _N // 2
_K2 = _K1 + 1

_NC, _NS, _L = 2, 16, 16
_NW = _NC * _NS
_RPW = (_B * _J * _H) // _NW
_CROWS = 64
_NCHUNKS = _RPW // _CROWS
_VPR = _W // _L

_BINS12 = 2048
_BINS3 = 1024

_mesh = plsc.VectorSubcoreMesh(
    core_axis_name="c", subcore_axis_name="s", num_cores=_NC, num_subcores=_NS
)


_MR = _B * _J * _H
_BB = 4
_BLKR = _BB * _J * _H


def _loss_body(o_ref, t_ref, w_ref, out_ref):
    w = w_ref[...][..., None]
    d = o_ref[...] * w - t_ref[...] * w
    bits = lax.bitcast_convert_type(0.5 * (d * d), jnp.int32)
    out_ref[...] = bits.reshape(_BLKR, _W)


def _loss_tc(o4, t4, w3):
    return pl.pallas_call(
        _loss_body,
        grid=(_B // _BB,),
        in_specs=[
            pl.BlockSpec((_BB, _J, _H, _W), lambda i: (i, 0, 0, 0)),
            pl.BlockSpec((_BB, _J, _H, _W), lambda i: (i, 0, 0, 0)),
            pl.BlockSpec((_BB, _J, 1), lambda i: (i, 0, 0)),
        ],
        out_specs=pl.BlockSpec((_BLKR, _W), lambda i: (i, 0)),
        out_shape=jax.ShapeDtypeStruct((_MR, _W), jnp.int32),
    )(o4, t4, w3)


def _make_sc_pass(npass):
    nbins = _BINS12 if npass < 3 else _BINS3
    nsec = 1 if npass == 1 else 2
    stride = nbins + 1
    hw = nsec * _L * stride
    bin_shift = {1: 21, 2: 10, 3: 0}[npass]
    pref_shift = {2: 21, 3: 10}.get(npass)

    scratch = [
        pltpu.VMEM((_CROWS, _W), jnp.int32),
        pltpu.VMEM((_CROWS, _W), jnp.int32),
        pltpu.VMEM((hw,), jnp.int32),
        pltpu.VMEM((nsec * nbins,), jnp.int32),
        pltpu.SemaphoreType.DMA,
        pltpu.SemaphoreType.DMA,
    ]
    if npass > 1:
        scratch.append(pltpu.VMEM((1024,), jnp.int32))

    @functools.partial(
        pl.kernel,
        out_type=jax.ShapeDtypeStruct((_NW, nsec * nbins), jnp.int32),
        mesh=_mesh,
        scratch_types=scratch,
        compiler_params=pltpu.CompilerParams(needs_layout_passes=False),
    )
    def sc_hist(bits_hbm, *args):
        if npass > 1:
            sel_hbm, hist_hbm, buf0, buf1, hist_v, out_v, sem0, sem1, sel_v = args
        else:
            hist_hbm, buf0, buf1, hist_v, out_v, sem0, sem1 = args
        bufs = (buf0, buf1)
        sems = (sem0, sem1)
        cid = lax.axis_index("c")
        sid = lax.axis_index("s")
        wid = sid * _NC + cid
        lane = lax.iota(jnp.int32, _L)
        ones = jnp.ones((_L,), jnp.int32)
        zeros = jnp.zeros((_L,), jnp.int32)

        def zero_body(i, carry):
            hist_v[pl.ds(i * _L, _L)] = zeros
            return carry

        lax.fori_loop(0, hw // _L, zero_body, 0)

        if npass > 1:
            pltpu.sync_copy(sel_hbm, sel_v)
            sel_a = sel_v[pl.ds(0, _L)]
            sel_b = sel_v[pl.ds(128, _L)]

        lane_base = lane * stride
        w_row0 = wid * _RPW

        def copy_desc(ci, b):
            return pltpu.make_async_copy(
                bits_hbm.at[pl.ds(w_row0 + ci * _CROWS, _CROWS), :],
                bufs[b],
                sems[b],
            )

        def process(buf):
            def row_body(r, c2):
                for u in range(_VPR):
                    v = buf[r, pl.ds(u * _L, _L)]
                    bins = jnp.right_shift(v, bin_shift) & (nbins - 1)
                    idx = lane_base + bins
                    if npass == 1:
                        plsc.addupdate_scatter(hist_v, [idx], ones)
                    else:
                        pref = jnp.right_shift(v, pref_shift)
                        plsc.addupdate_scatter(
                            hist_v, [idx], ones, mask=pref == sel_a
                        )
                        plsc.addupdate_scatter(
                            hist_v, [idx + _L * stride], ones, mask=pref == sel_b
                        )
                return c2

            lax.fori_loop(0, _CROWS, row_body, 0)

        copy_desc(0, 0).start()
        copy_desc(1, 1).start()

        def chunk_body(g, carry):
            for b in range(2):
                ci = g * 2 + b
                copy_desc(ci, b).wait()
                process(bufs[b])
                copy_desc(ci + 2, b).start()
            return carry

        lax.fori_loop(0, _NCHUNKS // 2 - 1, chunk_body, 0)
        for b in range(2):
            ci = _NCHUNKS - 2 + b
            copy_desc(ci, b).wait()
            process(bufs[b])

        for s in range(nsec):
            def red_body(j, carry, s=s):
                acc = zeros
                for l in range(_L):
                    gidx = (s * _L + l) * stride + j * _L + lane
                    acc = acc + plsc.load_gather(hist_v, [gidx])
                out_v[pl.ds(s * nbins + j * _L, _L)] = acc
                return carry

            lax.fori_loop(0, nbins // _L, red_body, 0)
        pltpu.sync_copy(out_v, hist_hbm.at[wid])

    return sc_hist


_sc_pass1 = _make_sc_pass(1)
_sc_pass2 = _make_sc_pass(2)
_sc_pass3 = _make_sc_pass(3)


def _cumsum_lanes(x):
    m = x.shape[1]
    s = 1
    while s < m:
        x = x + jnp.concatenate(
            [jnp.zeros((1, s), x.dtype), x[:, : m - s]], axis=1
        )
        s *= 2
    return x


def _row_val(arr, row):
    r = lax.broadcasted_iota(jnp.int32, arr.shape, 0)
    return jnp.max(jnp.where(r == row, arr, 0))


def _pack_rows(vals, dtype):
    r = lax.broadcasted_iota(jnp.int32, (8, 128), 0)
    out = jnp.zeros((8, 128), dtype)
    for i, v in enumerate(vals):
        out = jnp.where(r == i, v, out)
    return out


def _find(cum, rank):
    below = cum < rank
    b = jnp.sum(below.astype(jnp.int32))
    cnt = jnp.max(jnp.where(below, cum, 0))
    return b, cnt


def _sel1_body(h_ref, out_ref):
    h = h_ref[...]
    tot = jnp.sum(h, axis=0, keepdims=True)
    cum = _cumsum_lanes(tot)
    bin_a, cnt_a = _find(cum, _K1)
    bin_b, cnt_b = _find(cum, _K2)
    out_ref[...] = _pack_rows([bin_a, bin_b, cnt_a, cnt_b], jnp.int32)


def _sel23_body(h_ref, sel_ref, out_ref, *, nbins, final):
    h = h_ref[...]
    tot = jnp.sum(h, axis=0, keepdims=True)
    cum_a = _cumsum_lanes(tot[:, :nbins])
    cum_b = _cumsum_lanes(tot[:, nbins:])
    sel = sel_ref[...]
    pref_a = _row_val(sel, 0)
    pref_b = _row_val(sel, 1)
    cnt_a = _row_val(sel, 2)
    cnt_b = _row_val(sel, 3)
    bin2_a, cnt2_a = _find(cum_a, _K1 - cnt_a)
    bin2_b, cnt2_b = _find(cum_b, _K2 - cnt_b)
    npref_a = pref_a * nbins + bin2_a
    npref_b = pref_b * nbins + bin2_b
    if not final:
        out_ref[...] = _pack_rows(
            [npref_a, npref_b, cnt_a + cnt2_a, cnt_b + cnt2_b], jnp.int32
        )
    else:
        a = lax.bitcast_convert_type(
            jnp.full((1, 1), npref_a, jnp.int32), jnp.float32
        )
        b = lax.bitcast_convert_type(
            jnp.full((1, 1), npref_b, jnp.int32), jnp.float32
        )
        med = jnp.broadcast_to((a + b) * 0.5, (8, 128))
        r = lax.broadcasted_iota(jnp.int32, (8, 128), 0)
        out_ref[...] = jnp.where(r == 0, med, 0.0)


def _select1(h):
    return pl.pallas_call(
        _sel1_body,
        out_shape=jax.ShapeDtypeStruct((8, 128), jnp.int32),
    )(h)


def _select23(h, sel, nbins, final):
    body = functools.partial(_sel23_body, nbins=nbins, final=final)
    return pl.pallas_call(
        body,
        out_shape=jax.ShapeDtypeStruct(
            (8, 128), jnp.float32 if final else jnp.int32
        ),
    )(h, sel)


def _final_body(loss_ref, med_ref, out_ref, acc_s, acc_c):
    i = pl.program_id(0)
    med = jnp.max(med_ref[...])
    x = lax.bitcast_convert_type(loss_ref[...], jnp.float32)
    m = x > med
    s = jnp.sum(jnp.where(m, x, 0.0))
    c = jnp.sum(m.astype(jnp.int32))

    @pl.when(i == 0)
    def _():
        acc_s[0] = s
        acc_c[0] = c

    @pl.when(i > 0)
    def _():
        acc_s[0] += s
        acc_c[0] += c

    @pl.when(i == pl.num_programs(0) - 1)
    def _():
        out_ref[...] = jnp.full(
            (8, 128), acc_s[0] / acc_c[0].astype(jnp.float32), jnp.float32
        )


def _final_tc(bits2, med):
    return pl.pallas_call(
        _final_body,
        grid=(_MR // _BLKR,),
        in_specs=[
            pl.BlockSpec((_BLKR, _W), lambda i: (i, 0)),
            pl.BlockSpec((8, 128), lambda i: (0, 0)),
        ],
        out_specs=pl.BlockSpec((8, 128), lambda i: (0, 0)),
        out_shape=jax.ShapeDtypeStruct((8, 128), jnp.float32),
        scratch_shapes=[
            pltpu.SMEM((1,), jnp.float32),
            pltpu.SMEM((1,), jnp.int32),
        ],
    )(bits2, med)


def kernel(output, target, target_weight):
    bits = _loss_tc(output, target, target_weight)

    h1 = _sc_pass1(bits)
    sel1 = _select1(h1)
    h2 = _sc_pass2(bits, sel1.reshape(-1))
    sel2 = _select23(h2, sel1, _BINS12, False)
    h3 = _sc_pass3(bits, sel2.reshape(-1))
    med = _select23(h3, sel2, _BINS3, True)

    res = _final_tc(bits, med)
    return res[0, 0]


# speedup vs baseline: 31.7151x; 1.0104x over previous
import functools

import jax
import jax.numpy as jnp
from jax import lax
from jax.experimental import pallas as pl
from jax.experimental.pallas import tpu as pltpu
from jax.experimental.pallas import tpu_sc as plsc

_B, _J, _H, _W = 128, 17, 96, 96
_HW = _H * _W
_ROWS = _B * _J
_N = _ROWS * _HW
_K1 = _N // 2
_K2 = _K1 + 1

_NC, _NS, _L = 2, 16, 16
_NW = _NC * _NS
_RPW = (_B * _J * _H) // _NW
_CROWS = 64
_NCHUNKS = _RPW // _CROWS
_VPR = _W // _L
_RUNROLL = 4

_BINS12 = 2048
_BINS3 = 1024

_mesh = plsc.VectorSubcoreMesh(
    core_axis_name="c", subcore_axis_name="s", num_cores=_NC, num_subcores=_NS
)


_MR = _B * _J * _H
_BB = 4
_BLKR = _BB * _J * _H


def _loss_body(o_ref, t_ref, w_ref, out_ref):
    w = w_ref[...][..., None]
    d = o_ref[...] * w - t_ref[...] * w
    bits = lax.bitcast_convert_type(0.5 * (d * d), jnp.int32)
    out_ref[...] = bits.reshape(_BLKR, _W)


def _loss_tc(o4, t4, w3):
    return pl.pallas_call(
        _loss_body,
        grid=(_B // _BB,),
        in_specs=[
            pl.BlockSpec((_BB, _J, _H, _W), lambda i: (i, 0, 0, 0)),
            pl.BlockSpec((_BB, _J, _H, _W), lambda i: (i, 0, 0, 0)),
            pl.BlockSpec((_BB, _J, 1), lambda i: (i, 0, 0)),
        ],
        out_specs=pl.BlockSpec((_BLKR, _W), lambda i: (i, 0)),
        out_shape=jax.ShapeDtypeStruct((_MR, _W), jnp.int32),
    )(o4, t4, w3)


def _make_sc_pass(npass):
    nbins = _BINS12 if npass < 3 else _BINS3
    nsec = 1 if npass == 1 else 2
    stride = nbins + 1
    hw = nsec * _L * stride
    bin_shift = {1: 21, 2: 10, 3: 0}[npass]
    pref_shift = {2: 21, 3: 10}.get(npass)

    scratch = [
        pltpu.VMEM((_CROWS, _W), jnp.int32),
        pltpu.VMEM((_CROWS, _W), jnp.int32),
        pltpu.VMEM((hw,), jnp.int32),
        pltpu.VMEM((nsec * nbins,), jnp.int32),
        pltpu.SemaphoreType.DMA,
        pltpu.SemaphoreType.DMA,
    ]
    if npass > 1:
        scratch.append(pltpu.VMEM((1024,), jnp.int32))

    @functools.partial(
        pl.kernel,
        out_type=jax.ShapeDtypeStruct((_NW, nsec * nbins), jnp.int32),
        mesh=_mesh,
        scratch_types=scratch,
        compiler_params=pltpu.CompilerParams(needs_layout_passes=False),
    )
    def sc_hist(bits_hbm, *args):
        if npass > 1:
            sel_hbm, hist_hbm, buf0, buf1, hist_v, out_v, sem0, sem1, sel_v = args
        else:
            hist_hbm, buf0, buf1, hist_v, out_v, sem0, sem1 = args
        bufs = (buf0, buf1)
        sems = (sem0, sem1)
        cid = lax.axis_index("c")
        sid = lax.axis_index("s")
        wid = sid * _NC + cid
        lane = lax.iota(jnp.int32, _L)
        ones = jnp.ones((_L,), jnp.int32)
        zeros = jnp.zeros((_L,), jnp.int32)

        def zero_body(i, carry):
            hist_v[pl.ds(i * _L, _L)] = zeros
            return carry

        lax.fori_loop(0, hw // _L, zero_body, 0)

        if npass > 1:
            pltpu.sync_copy(sel_hbm, sel_v)
            sel_a = sel_v[pl.ds(0, _L)]
            sel_b = sel_v[pl.ds(128, _L)]

        lane_base = lane * stride
        w_row0 = wid * _RPW

        def copy_desc(ci, b):
            return pltpu.make_async_copy(
                bits_hbm.at[pl.ds(w_row0 + ci * _CROWS, _CROWS), :],
                bufs[b],
                sems[b],
            )

        def process(buf):
            def row_body(rr, c2):
                for r2 in range(_RUNROLL):
                    r = rr * _RUNROLL + r2
                    for u in range(_VPR):
                        v = buf[r, pl.ds(u * _L, _L)]
                        bins = jnp.right_shift(v, bin_shift) & (nbins - 1)
                        idx = lane_base + bins
                        if npass == 1:
                            plsc.addupdate_scatter(hist_v, [idx], ones)
                        else:
                            pref = jnp.right_shift(v, pref_shift)
                            plsc.addupdate_scatter(
                                hist_v, [idx], ones, mask=pref == sel_a
                            )
                            plsc.addupdate_scatter(
                                hist_v,
                                [idx + _L * stride],
                                ones,
                                mask=pref == sel_b,
                            )
                return c2

            lax.fori_loop(0, _CROWS // _RUNROLL, row_body, 0)

        copy_desc(0, 0).start()
        copy_desc(1, 1).start()

        def chunk_body(g, carry):
            for b in range(2):
                ci = g * 2 + b
                copy_desc(ci, b).wait()
                process(bufs[b])
                copy_desc(ci + 2, b).start()
            return carry

        lax.fori_loop(0, _NCHUNKS // 2 - 1, chunk_body, 0)
        for b in range(2):
            ci = _NCHUNKS - 2 + b
            copy_desc(ci, b).wait()
            process(bufs[b])

        for s in range(nsec):
            def red_body(j, carry, s=s):
                acc = zeros
                for l in range(_L):
                    gidx = (s * _L + l) * stride + j * _L + lane
                    acc = acc + plsc.load_gather(hist_v, [gidx])
                out_v[pl.ds(s * nbins + j * _L, _L)] = acc
                return carry

            lax.fori_loop(0, nbins // _L, red_body, 0)
        pltpu.sync_copy(out_v, hist_hbm.at[wid])

    return sc_hist


_sc_pass1 = _make_sc_pass(1)
_sc_pass2 = _make_sc_pass(2)
_sc_pass3 = _make_sc_pass(3)


def _cumsum_lanes(x):
    m = x.shape[1]
    s = 1
    while s < m:
        x = x + jnp.concatenate(
            [jnp.zeros((1, s), x.dtype), x[:, : m - s]], axis=1
        )
        s *= 2
    return x


def _row_val(arr, row):
    r = lax.broadcasted_iota(jnp.int32, arr.shape, 0)
    return jnp.max(jnp.where(r == row, arr, 0))


def _pack_rows(vals, dtype):
    r = lax.broadcasted_iota(jnp.int32, (8, 128), 0)
    out = jnp.zeros((8, 128), dtype)
    for i, v in enumerate(vals):
        out = jnp.where(r == i, v, out)
    return out


def _find(cum, rank):
    below = cum < rank
    b = jnp.sum(below.astype(jnp.int32))
    cnt = jnp.max(jnp.where(below, cum, 0))
    return b, cnt


def _sel1_body(h_ref, out_ref):
    h = h_ref[...]
    tot = jnp.sum(h, axis=0, keepdims=True)
    cum = _cumsum_lanes(tot)
    bin_a, cnt_a = _find(cum, _K1)
    bin_b, cnt_b = _find(cum, _K2)
    out_ref[...] = _pack_rows([bin_a, bin_b, cnt_a, cnt_b], jnp.int32)


def _sel23_body(h_ref, sel_ref, out_ref, *, nbins, final):
    h = h_ref[...]
    tot = jnp.sum(h, axis=0, keepdims=True)
    cum_a = _cumsum_lanes(tot[:, :nbins])
    cum_b = _cumsum_lanes(tot[:, nbins:])
    sel = sel_ref[...]
    pref_a = _row_val(sel, 0)
    pref_b = _row_val(sel, 1)
    cnt_a = _row_val(sel, 2)
    cnt_b = _row_val(sel, 3)
    bin2_a, cnt2_a = _find(cum_a, _K1 - cnt_a)
    bin2_b, cnt2_b = _find(cum_b, _K2 - cnt_b)
    npref_a = pref_a * nbins + bin2_a
    npref_b = pref_b * nbins + bin2_b
    if not final:
        out_ref[...] = _pack_rows(
            [npref_a, npref_b, cnt_a + cnt2_a, cnt_b + cnt2_b], jnp.int32
        )
    else:
        a = lax.bitcast_convert_type(
            jnp.full((1, 1), npref_a, jnp.int32), jnp.float32
        )
        b = lax.bitcast_convert_type(
            jnp.full((1, 1), npref_b, jnp.int32), jnp.float32
        )
        med = jnp.broadcast_to((a + b) * 0.5, (8, 128))
        r = lax.broadcasted_iota(jnp.int32, (8, 128), 0)
        out_ref[...] = jnp.where(r == 0, med, 0.0)


def _select1(h):
    return pl.pallas_call(
        _sel1_body,
        out_shape=jax.ShapeDtypeStruct((8, 128), jnp.int32),
    )(h)


def _select23(h, sel, nbins, final):
    body = functools.partial(_sel23_body, nbins=nbins, final=final)
    return pl.pallas_call(
        body,
        out_shape=jax.ShapeDtypeStruct(
            (8, 128), jnp.float32 if final else jnp.int32
        ),
    )(h, sel)


def _final_body(loss_ref, med_ref, out_ref, acc_s, acc_c):
    i = pl.program_id(0)
    med = jnp.max(med_ref[...])
    x = lax.bitcast_convert_type(loss_ref[...], jnp.float32)
    m = x > med
    s = jnp.sum(jnp.where(m, x, 0.0))
    c = jnp.sum(m.astype(jnp.int32))

    @pl.when(i == 0)
    def _():
        acc_s[0] = s
        acc_c[0] = c

    @pl.when(i > 0)
    def _():
        acc_s[0] += s
        acc_c[0] += c

    @pl.when(i == pl.num_programs(0) - 1)
    def _():
        out_ref[...] = jnp.full(
            (8, 128), acc_s[0] / acc_c[0].astype(jnp.float32), jnp.float32
        )


def _final_tc(bits2, med):
    return pl.pallas_call(
        _final_body,
        grid=(_MR // _BLKR,),
        in_specs=[
            pl.BlockSpec((_BLKR, _W), lambda i: (i, 0)),
            pl.BlockSpec((8, 128), lambda i: (0, 0)),
        ],
        out_specs=pl.BlockSpec((8, 128), lambda i: (0, 0)),
        out_shape=jax.ShapeDtypeStruct((8, 128), jnp.float32),
        scratch_shapes=[
            pltpu.SMEM((1,), jnp.float32),
            pltpu.SMEM((1,), jnp.int32),
        ],
    )(bits2, med)


def kernel(output, target, target_weight):
    bits = _loss_tc(output, target, target_weight)

    h1 = _sc_pass1(bits)
    sel1 = _select1(h1)
    h2 = _sc_pass2(bits, sel1.reshape(-1))
    sel2 = _select23(h2, sel1, _BINS12, False)
    h3 = _sc_pass3(bits, sel2.reshape(-1))
    med = _select23(h3, sel2, _BINS3, True)

    res = _final_tc(bits, med)
    return res[0, 0]
